# Initial kernel scaffold; baseline (speedup 1.0000x reference)
import functools
import math

import jax
import jax.numpy as jnp
from jax import lax
from jax.experimental import pallas as pl
from jax.experimental.pallas import tpu as pltpu
from jax.experimental.pallas import tpu_sc as plsc

_B, _L, _OMEGA = 4096, 200, 64
_VOCAB = 256
_MAX_SEQ_LEN = 512
_D = 2 * _OMEGA
_T = _B * _L

_LBLK = 8


def _expand_body(tab_ref, w_ref, b_ref, out_ref):
    t = tab_ref[...]
    z = lax.dot_general(t, w_ref[...], (((1,), (1,)), ((), ())),
                        preferred_element_type=jnp.float32)
    g = t * jax.nn.sigmoid(z + b_ref[...])
    i = pl.program_id(0)
    alpha = 2.0 * math.pi / _MAX_SEQ_LEN
    phi = alpha * (i * _LBLK + lax.broadcasted_iota(jnp.float32, (_LBLK, 1, 1), 0))
    out_ref[:, :, 0:_OMEGA] = g[None, :, :] * jnp.cos(phi)
    out_ref[:, :, _OMEGA:_D] = g[None, :, :] * jnp.sin(phi)


def _expand(table, W_gate, b_gate):
    return pl.pallas_call(
        _expand_body,
        grid=(_L // _LBLK,),
        in_specs=[
            pl.BlockSpec((_VOCAB, _OMEGA), lambda i: (0, 0)),
            pl.BlockSpec((_OMEGA, _OMEGA), lambda i: (0, 0)),
            pl.BlockSpec((1, _OMEGA), lambda i: (0, 0)),
        ],
        out_specs=pl.BlockSpec((_LBLK, _VOCAB, _D), lambda i: (i, 0, 0)),
        out_shape=jax.ShapeDtypeStruct((_L, _VOCAB, _D), jnp.float32),
    )(table, W_gate, b_gate.reshape(1, _OMEGA))


_NC, _NS = 2, 16
_NW = _NC * _NS
_TPW = _T // _NW
_C = 256
_H = 128
_NCHUNK = _TPW // _C

_mesh = plsc.VectorSubcoreMesh(core_axis_name="c", subcore_axis_name="s")


@functools.partial(
    pl.kernel,
    mesh=_mesh,
    out_type=jax.ShapeDtypeStruct((_T, _D), jnp.float32),
    scratch_types=[
        pltpu.VMEM((_C,), jnp.int32),
        pltpu.VMEM((_C,), jnp.int32),
        pltpu.VMEM((_H,), jnp.int32),
        pltpu.VMEM((_H,), jnp.int32),
        pltpu.VMEM((_H,), jnp.int32),
        pltpu.VMEM((_H,), jnp.int32),
        pltpu.VMEM((_C, _D), jnp.float32),
        pltpu.VMEM((_C, _D), jnp.float32),
        pltpu.SemaphoreType.DMA,
        pltpu.SemaphoreType.DMA,
    ],
)
def _lookup(x_hbm, g_hbm, out_hbm,
            xb0, xb1, ib00, ib01, ib10, ib11, rb0, rb1, sem0, sem1):
    xb = (xb0, xb1)
    ib = ((ib00, ib01), (ib10, ib11))
    rb = (rb0, rb1)
    sem = (sem0, sem1)
    wid = lax.axis_index("s") * _NC + lax.axis_index("c")
    base = wid * _TPW

    def stage(k, s):
        off = base + k * _C
        pltpu.sync_copy(x_hbm.at[pl.ds(off, _C)], xb[s])
        for h in range(_C // _H):
            for j in range(_H // 16):
                jj = h * _H + j * 16
                tvec = off + jj + lax.iota(jnp.int32, 16)
                ib[s][h][pl.ds(j * 16, 16)] = (
                    xb[s][pl.ds(jj, 16)] + (tvec % _L) * _VOCAB)
            pltpu.async_copy(g_hbm.at[ib[s][h]],
                             rb[s].at[pl.ds(h * _H, _H)], sem[s])

    def drain(k, s):
        for h in range(_C // _H):
            pltpu.make_async_copy(g_hbm.at[ib[s][h]],
                                  rb[s].at[pl.ds(h * _H, _H)], sem[s]).wait()
        pltpu.sync_copy(rb[s], out_hbm.at[pl.ds(base + k * _C, _C)])

    stage(0, 0)

    def body(k2, carry):
        for b in range(2):
            k = k2 * 2 + b

            @pl.when(k + 1 < _NCHUNK)
            def _():
                stage(k + 1, (b + 1) % 2)

            drain(k, b)
        return carry

    lax.fori_loop(0, _NCHUNK // 2, body, 0)


def kernel(x, table, W_gate, b_gate):
    G = _expand(table, W_gate, b_gate).reshape(_L * _VOCAB, _D)
    out = _lookup(x.reshape(_T), G)
    return out.reshape(_B, _L, _D)


# speedup vs baseline: 10.6021x; 10.6021x over previous
import functools
import math

import jax
import jax.numpy as jnp
from jax import lax
from jax.experimental import pallas as pl
from jax.experimental.pallas import tpu as pltpu
from jax.experimental.pallas import tpu_sc as plsc

_B, _L, _OMEGA = 4096, 200, 64
_VOCAB = 256
_MAX_SEQ_LEN = 512
_D = 2 * _OMEGA
_T = _B * _L

_LBLK = 8


def _expand_body(tab_ref, w_ref, b_ref, out_ref):
    t = tab_ref[...]
    z = lax.dot_general(t, w_ref[...], (((1,), (1,)), ((), ())),
                        preferred_element_type=jnp.float32)
    g = t * jax.nn.sigmoid(z + b_ref[...])
    i = pl.program_id(0)
    alpha = 2.0 * math.pi / _MAX_SEQ_LEN
    pos = i * _LBLK + lax.broadcasted_iota(jnp.int32, (_LBLK, 1, 1), 0)
    phi = alpha * pos.astype(jnp.float32)
    out_ref[:, :, 0:_OMEGA] = g[None, :, :] * jnp.cos(phi)
    out_ref[:, :, _OMEGA:_D] = g[None, :, :] * jnp.sin(phi)


def _expand(table, W_gate, b_gate):
    return pl.pallas_call(
        _expand_body,
        grid=(_L // _LBLK,),
        in_specs=[
            pl.BlockSpec((_VOCAB, _OMEGA), lambda i: (0, 0)),
            pl.BlockSpec((_OMEGA, _OMEGA), lambda i: (0, 0)),
            pl.BlockSpec((1, _OMEGA), lambda i: (0, 0)),
        ],
        out_specs=pl.BlockSpec((_LBLK, _VOCAB, _D), lambda i: (i, 0, 0)),
        out_shape=jax.ShapeDtypeStruct((_L, _VOCAB, _D), jnp.float32),
    )(table, W_gate, b_gate.reshape(1, _OMEGA))


_NC, _NS = 2, 16
_NW = _NC * _NS
_TPW = _T // _NW
_C = 256
_H = 128
_NCHUNK = _TPW // _C

@functools.cache
def _build_lookup():
    mesh = plsc.VectorSubcoreMesh(core_axis_name="c", subcore_axis_name="s")
    return functools.partial(
        pl.kernel,
        mesh=mesh,
        out_type=jax.ShapeDtypeStruct((_T, _D), jnp.float32),
        scratch_types=[
            pltpu.VMEM((_C,), jnp.int32),
            pltpu.VMEM((_C,), jnp.int32),
            pltpu.VMEM((_H,), jnp.int32),
            pltpu.VMEM((_H,), jnp.int32),
            pltpu.VMEM((_H,), jnp.int32),
            pltpu.VMEM((_H,), jnp.int32),
            pltpu.VMEM((_C, _D), jnp.float32),
            pltpu.VMEM((_C, _D), jnp.float32),
            pltpu.SemaphoreType.DMA,
            pltpu.SemaphoreType.DMA,
        ],
    )(_lookup_body)


def _lookup_body(x_hbm, g_hbm, out_hbm,
            xb0, xb1, ib00, ib01, ib10, ib11, rb0, rb1, sem0, sem1):
    xb = (xb0, xb1)
    ib = ((ib00, ib01), (ib10, ib11))
    rb = (rb0, rb1)
    sem = (sem0, sem1)
    wid = lax.axis_index("s") * _NC + lax.axis_index("c")
    base = wid * _TPW

    def stage(k, s):
        off = base + k * _C
        pltpu.sync_copy(x_hbm.at[pl.ds(off, _C)], xb[s])
        for h in range(_C // _H):
            for j in range(_H // 16):
                jj = h * _H + j * 16
                tvec = off + jj + lax.iota(jnp.int32, 16)
                ib[s][h][pl.ds(j * 16, 16)] = (
                    xb[s][pl.ds(jj, 16)] + (tvec % _L) * _VOCAB)
            pltpu.async_copy(g_hbm.at[ib[s][h]],
                             rb[s].at[pl.ds(h * _H, _H)], sem[s])

    def drain(k, s):
        for h in range(_C // _H):
            pltpu.make_async_copy(g_hbm.at[ib[s][h]],
                                  rb[s].at[pl.ds(h * _H, _H)], sem[s]).wait()
        pltpu.sync_copy(rb[s], out_hbm.at[pl.ds(base + k * _C, _C)])

    stage(0, 0)

    def body(k2, carry):
        for b in range(2):
            k = k2 * 2 + b

            @pl.when(k + 1 < _NCHUNK)
            def _():
                stage(k + 1, (b + 1) % 2)

            drain(k, b)
        return carry

    lax.fori_loop(0, _NCHUNK // 2, body, 0)


def kernel(x, table, W_gate, b_gate):
    G = _expand(table, W_gate, b_gate).reshape(_L * _VOCAB, _D)
    out = _build_lookup()(x.reshape(_T), G)
    return out.reshape(_B, _L, _D)
